# baseline (speedup 1.0000x reference)
import jax
import jax.numpy as jnp
from jax import lax
from jax.experimental import pallas as pl
from jax.experimental.pallas import tpu as pltpu
from jax.experimental.pallas import tpu_sc as plsc

NC = 2
NS = 16
NW = NC * NS
CHUNK = 128
BBLK = 2048


def _sc_gather(fine_W, coarse_W, fids3, cids3, B, D):
    b_per_w = B // NW
    n_chunks = b_per_w // CHUNK
    mesh = plsc.VectorSubcoreMesh(core_axis_name="c", subcore_axis_name="s",
                                  num_cores=NC, num_subcores=NS)

    def body(fine_W_hbm, coarse_W_hbm, fids_hbm, cids_hbm,
             fine_out, coarse_out, fidx, cidx, frows, crows, fsem, csem):
        wid = lax.axis_index("s") * NC + lax.axis_index("c")
        base = wid * b_per_w
        pltpu.sync_copy(fids_hbm.at[wid], fidx)
        pltpu.sync_copy(cids_hbm.at[wid], cidx)
        copies = []
        for j in range(n_chunks):
            copies.append(pltpu.async_copy(
                fine_W_hbm.at[fidx.at[j]],
                frows.at[pl.ds(j * CHUNK, CHUNK)], fsem))
        for j in range(n_chunks):
            copies.append(pltpu.async_copy(
                coarse_W_hbm.at[cidx.at[j]],
                crows.at[pl.ds(j * CHUNK, CHUNK)], csem))
        for cp in copies:
            cp.wait()
        pltpu.sync_copy(frows, fine_out.at[pl.ds(base, b_per_w)])
        pltpu.sync_copy(crows, coarse_out.at[pl.ds(base, b_per_w)])

    fn = pl.kernel(
        body,
        out_type=[jax.ShapeDtypeStruct((B, D), jnp.float32),
                  jax.ShapeDtypeStruct((B, D), jnp.float32)],
        mesh=mesh,
        scratch_types=[
            pltpu.VMEM((n_chunks, CHUNK), jnp.int32),
            pltpu.VMEM((n_chunks, CHUNK), jnp.int32),
            pltpu.VMEM((b_per_w, D), jnp.float32),
            pltpu.VMEM((b_per_w, D), jnp.float32),
            pltpu.SemaphoreType.DMA,
            pltpu.SemaphoreType.DMA,
        ],
        compiler_params=pltpu.CompilerParams(use_tc_tiling_on_sc=False),
    )
    return fn(fine_W, coarse_W, fids3, cids3)


def _tc_body(f_ref, c_ref, w1f_ref, w1c_ref, b1_ref, w2_ref, b2_ref,
             fused_ref, gate_ref):
    f = f_ref[...]
    c = c_ref[...]
    h = jnp.dot(f, w1f_ref[...], preferred_element_type=jnp.float32)
    h = h + jnp.dot(c, w1c_ref[...], preferred_element_type=jnp.float32)
    h = jnp.maximum(h + b1_ref[...], 0.0)
    z = jnp.dot(h, w2_ref[...], preferred_element_type=jnp.float32)
    g = 1.0 / (1.0 + jnp.exp(-(z + b2_ref[...])))
    fused_ref[...] = g * f + (1.0 - g) * c
    gate_ref[...] = g


def kernel(fine_ids, coarse_ids, fine_W, coarse_W, W1, b1, W2, b2):
    B = fine_ids.shape[0]
    D = fine_W.shape[1]
    b_per_w = B // NW
    n_chunks = b_per_w // CHUNK
    fids3 = fine_ids.reshape(NW, n_chunks, CHUNK)
    cids3 = coarse_ids.reshape(NW, n_chunks, CHUNK)

    fine_emb, coarse_emb = _sc_gather(fine_W, coarse_W, fids3, cids3, B, D)

    w1f = W1[:, :D].T
    w1c = W1[:, D:].T
    b1r = b1.reshape(1, -1)
    w2t = W2.T
    b2r = b2.reshape(1, 1)

    gh = W1.shape[0]
    grid = (B // BBLK,)
    fused, gate = pl.pallas_call(
        _tc_body,
        grid=grid,
        in_specs=[
            pl.BlockSpec((BBLK, D), lambda i: (i, 0)),
            pl.BlockSpec((BBLK, D), lambda i: (i, 0)),
            pl.BlockSpec((D, gh), lambda i: (0, 0)),
            pl.BlockSpec((D, gh), lambda i: (0, 0)),
            pl.BlockSpec((1, gh), lambda i: (0, 0)),
            pl.BlockSpec((gh, 1), lambda i: (0, 0)),
            pl.BlockSpec((1, 1), lambda i: (0, 0)),
        ],
        out_specs=[
            pl.BlockSpec((BBLK, D), lambda i: (i, 0)),
            pl.BlockSpec((BBLK, 1), lambda i: (i, 0)),
        ],
        out_shape=[
            jax.ShapeDtypeStruct((B, D), jnp.float32),
            jax.ShapeDtypeStruct((B, 1), jnp.float32),
        ],
    )(fine_emb, coarse_emb, w1f, w1c, b1r, w2t, b2r)
    return (fused, gate)


# speedup vs baseline: 1.0204x; 1.0204x over previous
import jax
import jax.numpy as jnp
from jax import lax
from jax.experimental import pallas as pl
from jax.experimental.pallas import tpu as pltpu
from jax.experimental.pallas import tpu_sc as plsc

NC = 2
NS = 16
NW = NC * NS
WLEN = 512
CAP = 4096
BBLK = 2048


def _sc_scan_gather(fine_WT, tail_pad, coarse_pT, ids, cids, B, D, V):
    n_reg = (V - 65) // WLEN
    n_win = n_reg + 1
    b_per_w = B // NW
    mesh = plsc.VectorSubcoreMesh(core_axis_name="c", subcore_axis_name="s",
                                  num_cores=NC, num_subcores=NS)

    def body(fine_hbm, tail_hbm, cpad_hbm, ids_hbm, cids_hbm,
             out_f, out_c,
             idsv, cidsv, midx, mval, qb, qcol, win0, win1, stage,
             wsem, osem):
        wid = lax.axis_index("s") * NC + lax.axis_index("c")
        n_base = n_win // NW
        n_extra = n_win - n_base * NW
        lo_w = wid * n_base + jnp.minimum(wid, n_extra)
        n_w = n_base + jnp.where(wid < n_extra, 1, 0)
        hi_w = lo_w + n_w
        base_b = wid * b_per_w

        pltpu.sync_copy(ids_hbm, idsv)
        pltpu.sync_copy(cids_hbm.at[pl.ds(base_b, b_per_w)], cidsv)

        e16 = lax.iota(jnp.int32, 16)

        lo_v = lo_w * WLEN
        hi_v = hi_w * WLEN

        def prefilter(g, cnt):
            v = idsv[pl.ds(g * 16, 16)]
            m = (v >= lo_v) & (v < hi_v)
            pos = jnp.minimum(cnt + plsc.cumsum(m.astype(jnp.int32)) - 1,
                              CAP - 1)
            plsc.store_scatter(midx, [pos], e16 + g * 16, mask=m)
            plsc.store_scatter(mval, [pos], v, mask=m)
            return cnt + plsc.all_reduce_population_count(m)[0]

        cnt = lax.fori_loop(0, B // 16, prefilter, 0)
        cnt = jnp.minimum(cnt, CAP)

        def win_dma(w, buf):
            @pl.when(w < n_reg)
            def _():
                c0 = pl.multiple_of(w * WLEN, 128)
                pltpu.async_copy(fine_hbm.at[:, pl.ds(c0, WLEN)], buf, wsem)

            @pl.when(w == n_reg)
            def _():
                pltpu.async_copy(tail_hbm, buf.at[:, pl.ds(0, 128)], wsem)

        def win_wait(w):
            @pl.when(w < n_reg)
            def _():
                pltpu.make_async_copy(
                    fine_hbm.at[:, pl.ds(0, WLEN)], win0, wsem).wait()

            @pl.when(w == n_reg)
            def _():
                pltpu.make_async_copy(
                    tail_hbm, win0.at[:, pl.ds(0, 128)], wsem).wait()

        def process_window(buf, h, my_cnt, base_v):
            def scan(q, qc):
                v = mval[pl.ds(q * 16, 16)]
                b = midx[pl.ds(q * 16, 16)]
                m = ((v >= base_v) & (v < base_v + WLEN)
                     & ((q * 16 + e16) < my_cnt))
                pos = jnp.minimum(qc + plsc.cumsum(m.astype(jnp.int32)) - 1,
                                  CAP - 1)
                plsc.store_scatter(qb, [pos], b, mask=m)
                plsc.store_scatter(qcol, [pos], v - base_v, mask=m)
                return qc + plsc.all_reduce_population_count(m)[0]

            n_groups = (my_cnt + 15) // 16
            qcnt = lax.fori_loop(0, n_groups, scan, 0)

            def chunk(qg, h):
                bv = qb[pl.ds(qg * 16, 16)]
                cv = qcol[pl.ds(qg * 16, 16)]
                valid = (e16 + qg * 16) < qcnt
                bv = jnp.where(valid, bv, B)
                cv = jnp.where(valid, cv, 0)
                r = (h // 16) % 4

                @pl.when(h >= 64)
                def _():
                    pltpu.make_async_copy(
                        stage.at[pl.ds(0, 16 * D)],
                        out_f.at[pl.ds(0, 16 * D)], osem).wait()

                for k in range(16):
                    colk = jnp.full((16,), cv[k], jnp.int32)
                    s0 = r * (16 * D) + k * D
                    for e0 in range(0, D, 16):
                        g = plsc.load_gather(buf, [e16 + e0, colk])
                        stage[pl.ds(s0 + e0, 16)] = g
                    pltpu.async_copy(stage.at[pl.ds(s0, D)],
                                     out_f.at[pl.ds(bv[k] * D, D)], osem)
                return h + 16

            n_chunks = (qcnt + 15) // 16
            return lax.fori_loop(0, n_chunks, chunk, h)

        win_dma(lo_w, win0)

        def pair_step(i2, h):
            w = lo_w + 2 * i2
            win_wait(w)

            @pl.when(w + 1 < hi_w)
            def _():
                win_dma(w + 1, win1)

            h = process_window(win0, h, cnt, w * WLEN)

            def second(h):
                win_wait(w + 1)

                @pl.when(w + 2 < hi_w)
                def _():
                    win_dma(w + 2, win0)

                return process_window(win1, h, cnt, (w + 1) * WLEN)

            h = lax.cond(w + 1 < hi_w, second, lambda h: h, h)
            return h

        h = lax.fori_loop(0, (n_w + 1) // 2, pair_step, 0)

        pltpu.async_copy(cpad_hbm.at[:, pl.ds(0, WLEN)], win0, wsem)
        pltpu.async_copy(cpad_hbm.at[:, pl.ds(WLEN, WLEN)], win1, wsem)
        pltpu.make_async_copy(
            cpad_hbm.at[:, pl.ds(0, WLEN)], win0, wsem).wait()
        pltpu.make_async_copy(
            cpad_hbm.at[:, pl.ds(0, WLEN)], win1, wsem).wait()

        def cfill(g, _):
            mval[pl.ds(g * 16, 16)] = cidsv[pl.ds(g * 16, 16)]
            midx[pl.ds(g * 16, 16)] = base_b + g * 16 + e16
            return 0

        lax.fori_loop(0, b_per_w // 16, cfill, 0)

        def process_coarse(buf, h, base_v):
            def scan(q, qc):
                v = mval[pl.ds(q * 16, 16)]
                b = midx[pl.ds(q * 16, 16)]
                m = (v >= base_v) & (v < base_v + WLEN)
                pos = jnp.minimum(qc + plsc.cumsum(m.astype(jnp.int32)) - 1,
                                  CAP - 1)
                plsc.store_scatter(qb, [pos], b, mask=m)
                plsc.store_scatter(qcol, [pos], v - base_v, mask=m)
                return qc + plsc.all_reduce_population_count(m)[0]

            qcnt = lax.fori_loop(0, b_per_w // 16, scan, 0)

            def chunk(qg, h):
                bv = qb[pl.ds(qg * 16, 16)]
                cv = qcol[pl.ds(qg * 16, 16)]
                valid = (e16 + qg * 16) < qcnt
                bv = jnp.where(valid, bv, B)
                cv = jnp.where(valid, cv, 0)
                r = (h // 16) % 4

                @pl.when(h >= 64)
                def _():
                    pltpu.make_async_copy(
                        stage.at[pl.ds(0, 16 * D)],
                        out_f.at[pl.ds(0, 16 * D)], osem).wait()

                for k in range(16):
                    colk = jnp.full((16,), cv[k], jnp.int32)
                    s0 = r * (16 * D) + k * D
                    for e0 in range(0, D, 16):
                        g = plsc.load_gather(buf, [e16 + e0, colk])
                        stage[pl.ds(s0 + e0, 16)] = g
                    pltpu.async_copy(stage.at[pl.ds(s0, D)],
                                     out_c.at[pl.ds(bv[k] * D, D)], osem)
                return h + 16

            n_chunks = (qcnt + 15) // 16
            return lax.fori_loop(0, n_chunks, chunk, h)

        h = process_coarse(win0, h, 0)
        h = process_coarse(win1, h, WLEN)

        def drain(i, _):
            pltpu.make_async_copy(
                stage.at[pl.ds(0, D)], out_f.at[pl.ds(0, D)], osem).wait()
            return 0

        lax.fori_loop(0, jnp.minimum(h, 64), drain, 0)

    fn = pl.kernel(
        body,
        out_type=[jax.ShapeDtypeStruct(((B + 16) * D,), jnp.float32),
                  jax.ShapeDtypeStruct(((B + 16) * D,), jnp.float32)],
        mesh=mesh,
        scratch_types=[
            pltpu.VMEM((B,), jnp.int32),
            pltpu.VMEM((b_per_w,), jnp.int32),
            pltpu.VMEM((CAP,), jnp.int32),
            pltpu.VMEM((CAP,), jnp.int32),
            pltpu.VMEM((CAP,), jnp.int32),
            pltpu.VMEM((CAP,), jnp.int32),
            pltpu.VMEM((D, WLEN), jnp.float32),
            pltpu.VMEM((D, WLEN), jnp.float32),
            pltpu.VMEM((4 * 16 * D,), jnp.float32),
            pltpu.SemaphoreType.DMA,
            pltpu.SemaphoreType.DMA,
        ],
        compiler_params=pltpu.CompilerParams(needs_layout_passes=False),
    )
    return fn(fine_WT, tail_pad, coarse_pT, ids, cids)


def _tc_body(f_ref, c_ref, w1f_ref, w1c_ref, b1_ref, w2_ref, b2_ref,
             fused_ref, gate_ref):
    f = f_ref[...]
    c = c_ref[...]
    h = jnp.dot(f, w1f_ref[...], preferred_element_type=jnp.float32)
    h = h + jnp.dot(c, w1c_ref[...], preferred_element_type=jnp.float32)
    h = jnp.maximum(h + b1_ref[...], 0.0)
    z = jnp.dot(h, w2_ref[...], preferred_element_type=jnp.float32)
    g = 1.0 / (1.0 + jnp.exp(-(z + b2_ref[...])))
    fused_ref[...] = g * f + (1.0 - g) * c
    gate_ref[...] = g


def kernel(fine_ids, coarse_ids, fine_W, coarse_W, W1, b1, W2, b2):
    B = fine_ids.shape[0]
    D = fine_W.shape[1]
    V = fine_W.shape[0]
    n_reg = (V - 65) // WLEN

    fine_WT = fine_W.T
    tail_pad = jnp.pad(fine_WT[:, n_reg * WLEN:],
                       ((0, 0), (0, 128 - (V - n_reg * WLEN))))
    coarse_pT = jnp.pad(coarse_W,
                        ((0, 2 * WLEN - coarse_W.shape[0]), (0, 0))).T

    fine_flat, coarse_flat = _sc_scan_gather(
        fine_WT, tail_pad, coarse_pT, fine_ids, coarse_ids, B, D, V)
    fine_emb = fine_flat[:B * D].reshape(B, D)
    coarse_emb = coarse_flat[:B * D].reshape(B, D)

    w1f = W1[:, :D].T
    w1c = W1[:, D:].T
    b1r = b1.reshape(1, -1)
    w2t = W2.T
    b2r = b2.reshape(1, 1)

    gh = W1.shape[0]
    grid = (B // BBLK,)
    fused, gate = pl.pallas_call(
        _tc_body,
        grid=grid,
        in_specs=[
            pl.BlockSpec((BBLK, D), lambda i: (i, 0)),
            pl.BlockSpec((BBLK, D), lambda i: (i, 0)),
            pl.BlockSpec((D, gh), lambda i: (0, 0)),
            pl.BlockSpec((D, gh), lambda i: (0, 0)),
            pl.BlockSpec((1, gh), lambda i: (0, 0)),
            pl.BlockSpec((gh, 1), lambda i: (0, 0)),
            pl.BlockSpec((1, 1), lambda i: (0, 0)),
        ],
        out_specs=[
            pl.BlockSpec((BBLK, D), lambda i: (i, 0)),
            pl.BlockSpec((BBLK, 1), lambda i: (i, 0)),
        ],
        out_shape=[
            jax.ShapeDtypeStruct((B, D), jnp.float32),
            jax.ShapeDtypeStruct((B, 1), jnp.float32),
        ],
    )(fine_emb, coarse_emb, w1f, w1c, b1r, w2t, b2r)
    return (fused, gate)
